# Initial kernel scaffold; baseline (speedup 1.0000x reference)
import jax
import jax.numpy as jnp
from jax.experimental import pallas as pl


def kernel(x_user, x_item, edge_buys, edge_bought, edge_follows, params):
    raise NotImplementedError("write your pallas kernel here")


# speedup vs baseline: 7.3203x; 7.3203x over previous
import functools

import jax
import jax.numpy as jnp
from jax import lax
from jax.experimental import pallas as pl
from jax.experimental.pallas import tpu as pltpu
from jax.experimental.pallas import tpu_sc as plsc

_H = 128
_NC = 2
_NS = 16
_NW = _NC * _NS
_C = 80
_ZR = 80


def _matmul_bias_body(x_ref, w_ref, b_ref, o_ref):
    o_ref[...] = (
        jnp.dot(x_ref[...], w_ref[...], preferred_element_type=jnp.float32)
        + b_ref[...]
    )


def _matmul_bias(x, w, b, block_n=1000):
    n, din = x.shape
    dout = w.shape[1]
    return pl.pallas_call(
        _matmul_bias_body,
        grid=(n // block_n,),
        in_specs=[
            pl.BlockSpec((block_n, din), lambda i: (i, 0)),
            pl.BlockSpec((din, dout), lambda i: (0, 0)),
            pl.BlockSpec((1, dout), lambda i: (0, 0)),
        ],
        out_specs=pl.BlockSpec((block_n, dout), lambda i: (i, 0)),
        out_shape=jax.ShapeDtypeStruct((n, dout), jnp.float32),
    )(x, w, b[None, :])


def _combine2_body(a0, a1, s0, c0, c1, s1, o):
    ya = a0[...] + a1[...] + s0[...]
    yb = c0[...] + c1[...] + s1[...]
    y = jnp.maximum(ya, yb)
    o[...] = jnp.where(y >= 0, y, 0.01 * y)


def _combine1_body(a0, a1, s0, o):
    y = a0[...] + a1[...] + s0[...]
    o[...] = jnp.where(y >= 0, y, 0.01 * y)


def _combine(parts, block_n=1000):
    n = parts[0][0].shape[0]
    args = [a for t in parts for a in t]
    body = _combine2_body if len(parts) == 2 else _combine1_body
    spec = pl.BlockSpec((block_n, _H), lambda i: (i, 0))
    return pl.pallas_call(
        body,
        grid=(n // block_n,),
        in_specs=[spec] * len(args),
        out_specs=spec,
        out_shape=jax.ShapeDtypeStruct((n, _H), jnp.float32),
    )(*args)


def _edge_body(src_ref, dst_ref, k_hbm, q_hbm, v_hbm, out_hbm,
               sidx, didx, kbuf, qbuf, vbuf, zbuf, acc, sem):
    c = lax.axis_index("c")
    s = lax.axis_index("s")
    wid = s * _NC + c
    n_dst = acc.shape[0]
    e_total = src_ref.shape[0]
    e_per_w = e_total // _NW
    n_chunks = e_per_w // _C
    n_pieces = n_dst // _ZR
    n_slots = (n_pieces + _NS - 1) // _NS

    def zrow(i, carry):
        for j in range(_H // 16):
            zbuf[i, pl.ds(j * 16, 16)] = jnp.zeros((16,), jnp.float32)
        return carry
    lax.fori_loop(0, _ZR, zrow, 0)
    for p in range(n_slots):
        pid = p * _NS + s
        r0 = pl.multiple_of(pid * _ZR, 8)

        @pl.when(pid < n_pieces)
        def _():
            pltpu.sync_copy(zbuf, acc.at[pl.ds(r0, _ZR)])
    plsc.subcore_barrier()

    def chunk(t, carry):
        base = pl.multiple_of(wid * e_per_w + t * _C, 8)
        pltpu.sync_copy(src_ref.at[pl.ds(base, _C)], sidx)
        pltpu.sync_copy(dst_ref.at[pl.ds(base, _C)], didx)
        d1 = pltpu.async_copy(k_hbm.at[didx], kbuf, sem)
        d2 = pltpu.async_copy(q_hbm.at[sidx], qbuf, sem)
        d3 = pltpu.async_copy(v_hbm.at[sidx], vbuf, sem)
        d1.wait()
        d2.wait()
        d3.wait()

        def row(r, rc):
            for j in range(_H // 16):
                sl = pl.ds(j * 16, 16)
                kx = kbuf[r, sl]
                qx = qbuf[r, sl]
                vx = vbuf[r, sl]
                g = 1.0 / (1.0 + jnp.exp(-(kx + qx)))
                vbuf[r, sl] = g * vx
            return rc
        lax.fori_loop(0, _C, row, 0)
        pltpu.sync_copy(vbuf, acc.at[didx], add=True)
        return carry
    lax.fori_loop(0, n_chunks, chunk, 0)
    plsc.subcore_barrier()

    for p in range(n_slots):
        pid = p * _NS + s
        r0 = pl.multiple_of(pid * _ZR, 8)

        @pl.when(pid < n_pieces)
        def _():
            pltpu.sync_copy(acc.at[pl.ds(r0, _ZR)], zbuf)
            pltpu.sync_copy(zbuf, out_hbm.at[c, pl.ds(r0, _ZR)])


def _edge_pass(src, dst, ktab, qtab, vtab, n_dst):
    mesh = plsc.VectorSubcoreMesh(core_axis_name="c", subcore_axis_name="s")
    f = pl.kernel(
        _edge_body,
        out_type=jax.ShapeDtypeStruct((_NC, n_dst, _H), jnp.float32),
        mesh=mesh,
        scratch_types=[
            pltpu.VMEM((_C,), jnp.int32),
            pltpu.VMEM((_C,), jnp.int32),
            pltpu.VMEM((_C, _H), jnp.float32),
            pltpu.VMEM((_C, _H), jnp.float32),
            pltpu.VMEM((_C, _H), jnp.float32),
            pltpu.VMEM((_ZR, _H), jnp.float32),
            pltpu.VMEM_SHARED((n_dst, _H), jnp.float32),
            pltpu.SemaphoreType.DMA,
        ],
    )
    return f(src, dst, ktab, qtab, vtab)


def kernel(x_user, x_item, edge_buys, edge_bought, edge_follows, params):
    eb = edge_buys.astype(jnp.int32)
    ebb = edge_bought.astype(jnp.int32)
    ef = edge_follows.astype(jnp.int32)
    n_user = x_user.shape[0]
    n_item = x_item.shape[0]
    xu, xi = x_user, x_item

    for l in range(len(params["layers"])):
        lp = params["layers"][l]
        pb, pf, pbb = lp["buys"], lp["follows"], lp["bought_by"]
        wu = jnp.concatenate(
            [pb["Wq"], pb["Wv"], pf["Wq"], pf["Wv"],
             pf["Wk"], pf["Ws"], pbb["Wk"], pbb["Ws"]], axis=1)
        bu = jnp.concatenate(
            [pb["bq"], pb["bv"], pf["bq"], pf["bv"],
             pf["bk"], pf["b"], pbb["bk"], pbb["b"]])
        wi = jnp.concatenate([pbb["Wq"], pbb["Wv"], pb["Wk"], pb["Ws"]], axis=1)
        bi = jnp.concatenate([pbb["bq"], pbb["bv"], pb["bk"], pb["b"]])
        up = _matmul_bias(xu, wu, bu)
        ip = _matmul_bias(xi, wi, bi)
        q_buys, v_buys = up[:, 0:128], up[:, 128:256]
        q_fol, v_fol = up[:, 256:384], up[:, 384:512]
        k_fol, s_fol = up[:, 512:640], up[:, 640:768]
        k_bought, s_bought = up[:, 768:896], up[:, 896:1024]
        q_bought, v_bought = ip[:, 0:128], ip[:, 128:256]
        k_buys, s_buys = ip[:, 256:384], ip[:, 384:512]

        agg_buys = _edge_pass(eb[0], eb[1], k_buys, q_buys, v_buys, n_item)
        agg_bought = _edge_pass(ebb[0], ebb[1], k_bought, q_bought, v_bought, n_user)
        agg_fol = _edge_pass(ef[0], ef[1], k_fol, q_fol, v_fol, n_user)

        xu = _combine([(agg_bought[0], agg_bought[1], s_bought),
                       (agg_fol[0], agg_fol[1], s_fol)])
        xi = _combine([(agg_buys[0], agg_buys[1], s_buys)])

    wp = jnp.pad(params["lin_W"], ((0, 0), (0, _H - params["lin_W"].shape[1])))
    bp = jnp.pad(params["lin_b"], (0, _H - params["lin_b"].shape[0]))
    out = _matmul_bias(xu, wp, bp)
    return out[:, :params["lin_W"].shape[1]]


# speedup vs baseline: 11.4213x; 1.5602x over previous
import jax
import jax.numpy as jnp
from jax import lax
from jax.experimental import pallas as pl
from jax.experimental.pallas import tpu as pltpu
from jax.experimental.pallas import tpu_sc as plsc

_H = 128
_NC = 2
_NS = 16
_NW = _NC * _NS
_C = 40
_NH = 5
_ZR = 40


def _matmul_bias_body(x_ref, w_ref, b_ref, o_ref):
    o_ref[...] = (
        jnp.dot(x_ref[...], w_ref[...], preferred_element_type=jnp.float32)
        + b_ref[...]
    )


def _matmul_bias(x, w, b, block_n=1000):
    n, din = x.shape
    dout = w.shape[1]
    return pl.pallas_call(
        _matmul_bias_body,
        grid=(n // block_n,),
        in_specs=[
            pl.BlockSpec((block_n, din), lambda i: (i, 0)),
            pl.BlockSpec((din, dout), lambda i: (0, 0)),
            pl.BlockSpec((1, dout), lambda i: (0, 0)),
        ],
        out_specs=pl.BlockSpec((block_n, dout), lambda i: (i, 0)),
        out_shape=jax.ShapeDtypeStruct((n, dout), jnp.float32),
    )(x, w, b[None, :])


def _combine2_body(a0, a1, s0, c0, c1, s1, o):
    ya = a0[...] + a1[...] + s0[...]
    yb = c0[...] + c1[...] + s1[...]
    y = jnp.maximum(ya, yb)
    o[...] = jnp.where(y >= 0, y, 0.01 * y)


def _combine1_body(a0, a1, s0, o):
    y = a0[...] + a1[...] + s0[...]
    o[...] = jnp.where(y >= 0, y, 0.01 * y)


def _combine(parts, block_n=1000):
    n = parts[0][0].shape[0]
    args = [a for t in parts for a in t]
    body = _combine2_body if len(parts) == 2 else _combine1_body
    spec = pl.BlockSpec((block_n, _H), lambda i: (i, 0))
    return pl.pallas_call(
        body,
        grid=(n // block_n,),
        in_specs=[spec] * len(args),
        out_specs=spec,
        out_shape=jax.ShapeDtypeStruct((n, _H), jnp.float32),
    )(*args)


def _edge_body(src_ref, dst_ref, k_hbm, qv_hbm, out_hbm,
               sidx, didx, kbuf0, kbuf1, kbuf2, kbuf3, qvbuf0, qvbuf1,
               dsc0, dsc1, dsc2, dsc3, acc,
               gsem0, gsem1, gsem2, gsem3,
               isem0, isem1, isem2, isem3,
               ssem0, ssem1, ssem2, ssem3):
    c = lax.axis_index("c")
    s = lax.axis_index("s")
    wid = s * _NC + c
    n_dst = acc.shape[0]
    e_res = sidx.shape[0]
    ch = e_res // _C
    assert ch % 4 == 0 and ch * _C == e_res
    kbufs = (kbuf0, kbuf1, kbuf2, kbuf3)
    qvbufs = (qvbuf0, qvbuf1)
    dscs = (dsc0, dsc1, dsc2, dsc3)
    gsems = (gsem0, gsem1, gsem2, gsem3)
    isems = (isem0, isem1, isem2, isem3)
    ssems = (ssem0, ssem1, ssem2, ssem3)

    def zrow(i, carry):
        for j in range(_H // 16):
            kbuf0[i, pl.ds(j * 16, 16)] = jnp.zeros((16,), jnp.float32)
        return carry
    lax.fori_loop(0, _ZR, zrow, 0)
    n_pieces = n_dst // _ZR
    n_slots = (n_pieces + _NS - 1) // _NS
    for p in range(n_slots):
        pid = p * _NS + s
        r0 = pl.multiple_of(pid * _ZR, 8)

        @pl.when(pid < n_pieces)
        def _():
            pltpu.sync_copy(kbuf0, acc.at[pl.ds(r0, _ZR)])
    plsc.subcore_barrier()

    def stretch(h, hcarry):
        gbase = pl.multiple_of((wid * _NH + h) * e_res, 8)
        pltpu.sync_copy(src_ref.at[pl.ds(gbase, e_res)], sidx)
        pltpu.sync_copy(dst_ref.at[pl.ds(gbase, e_res)], didx)

        def fire(t, ks, qs):
            off = pl.multiple_of(t * _C, 8)
            pltpu.async_copy(
                k_hbm.at[didx.at[pl.ds(off, _C)]], kbufs[ks], gsems[ks])
            pltpu.async_copy(
                qv_hbm.at[sidx.at[pl.ds(off, _C)]], qvbufs[qs], gsems[ks])
            hoff = pl.multiple_of(gbase + t * _C, 8)
            pltpu.async_copy(
                dst_ref.at[pl.ds(hoff, _C)], dscs[ks], isems[ks])

        fire(0, 0, 0)

        def quad(g, carry):
            t0 = g * 4
            for p in range(4):
                t = t0 + p
                pn = (p + 1) % 4
                tn = t + 1

                @pl.when(tn < ch)
                def _():
                    @pl.when(tn >= 4)
                    def _():
                        pltpu.make_async_copy(
                            k_hbm.at[pl.ds(0, _C)], kbufs[pn],
                            ssems[pn]).wait()
                    fire(tn, pn, (p + 1) % 2)

                pltpu.make_async_copy(
                    k_hbm.at[pl.ds(0, _C)], kbufs[p], gsems[p]).wait()
                pltpu.make_async_copy(
                    qv_hbm.at[pl.ds(0, _C)], qvbufs[p % 2], gsems[p]).wait()
                pltpu.make_async_copy(
                    dst_ref.at[pl.ds(0, _C)], dscs[p], isems[p]).wait()

                kb, qvb = kbufs[p], qvbufs[p % 2]

                @plsc.parallel_loop(0, _C)
                def row(r):
                    for j in range(_H // 16):
                        sl = pl.ds(j * 16, 16)
                        kx = kb[r, sl]
                        qx = qvb[r, sl]
                        vx = qvb[r, pl.ds(_H + j * 16, 16)]
                        g_ = 1.0 / (1.0 + jnp.exp(-(kx + qx)))
                        kb[r, sl] = g_ * vx

                pltpu.async_copy(kb, acc.at[dscs[p]], ssems[p], add=True)
            return carry
        lax.fori_loop(0, ch // 4, quad, 0)
        for p in range(4):
            pltpu.make_async_copy(
                k_hbm.at[pl.ds(0, _C)], kbufs[p], ssems[p]).wait()
        return hcarry
    lax.fori_loop(0, _NH, stretch, 0)
    plsc.subcore_barrier()

    for p in range(n_slots):
        pid = p * _NS + s
        r0 = pl.multiple_of(pid * _ZR, 8)

        @pl.when(pid < n_pieces)
        def _():
            pltpu.sync_copy(acc.at[pl.ds(r0, _ZR)], kbuf0)
            pltpu.sync_copy(kbuf0, out_hbm.at[c, pl.ds(r0, _ZR)])


def _edge_pass(src, dst, ktab, qvtab, n_dst):
    e_total = src.shape[0]
    e_res = e_total // (_NW * _NH)
    mesh = plsc.VectorSubcoreMesh(core_axis_name="c", subcore_axis_name="s")
    f = pl.kernel(
        _edge_body,
        out_type=jax.ShapeDtypeStruct((_NC, n_dst, _H), jnp.float32),
        mesh=mesh,
        scratch_types=(
            [pltpu.VMEM((e_res,), jnp.int32)] * 2
            + [pltpu.VMEM((_C, _H), jnp.float32)] * 4
            + [pltpu.VMEM((_C, 2 * _H), jnp.float32)] * 2
            + [pltpu.VMEM((_C,), jnp.int32)] * 4
            + [pltpu.VMEM_SHARED((n_dst, _H), jnp.float32)]
            + [pltpu.SemaphoreType.DMA] * 12
        ),
    )
    return f(src, dst, ktab, qvtab)


def kernel(x_user, x_item, edge_buys, edge_bought, edge_follows, params):
    eb = edge_buys.astype(jnp.int32)
    ebb = edge_bought.astype(jnp.int32)
    ef = edge_follows.astype(jnp.int32)
    n_user = x_user.shape[0]
    n_item = x_item.shape[0]
    xu, xi = x_user, x_item

    for l in range(len(params["layers"])):
        lp = params["layers"][l]
        pb, pf, pbb = lp["buys"], lp["follows"], lp["bought_by"]
        wu = jnp.concatenate(
            [pb["Wq"], pb["Wv"], pf["Wq"], pf["Wv"],
             pf["Wk"], pf["Ws"], pbb["Wk"], pbb["Ws"]], axis=1)
        bu = jnp.concatenate(
            [pb["bq"], pb["bv"], pf["bq"], pf["bv"],
             pf["bk"], pf["b"], pbb["bk"], pbb["b"]])
        wi = jnp.concatenate([pbb["Wq"], pbb["Wv"], pb["Wk"], pb["Ws"]], axis=1)
        bi = jnp.concatenate([pbb["bq"], pbb["bv"], pb["bk"], pb["b"]])
        up = _matmul_bias(xu, wu, bu)
        ip = _matmul_bias(xi, wi, bi)
        qv_buys = up[:, 0:256]
        qv_fol = up[:, 256:512]
        k_fol, s_fol = up[:, 512:640], up[:, 640:768]
        k_bought, s_bought = up[:, 768:896], up[:, 896:1024]
        qv_bought = ip[:, 0:256]
        k_buys, s_buys = ip[:, 256:384], ip[:, 384:512]

        agg_buys = _edge_pass(eb[0], eb[1], k_buys, qv_buys, n_item)
        agg_bought = _edge_pass(ebb[0], ebb[1], k_bought, qv_bought, n_user)
        agg_fol = _edge_pass(ef[0], ef[1], k_fol, qv_fol, n_user)

        xu = _combine([(agg_bought[0], agg_bought[1], s_bought),
                       (agg_fol[0], agg_fol[1], s_fol)])
        xi = _combine([(agg_buys[0], agg_buys[1], s_buys)])

    wp = jnp.pad(params["lin_W"], ((0, 0), (0, _H - params["lin_W"].shape[1])))
    bp = jnp.pad(params["lin_b"], (0, _H - params["lin_b"].shape[0]))
    out = _matmul_bias(xu, wp, bp)
    return out[:, :params["lin_W"].shape[1]]


# speedup vs baseline: 18.2181x; 1.5951x over previous
import jax
import jax.numpy as jnp
from jax import lax
from jax.experimental import pallas as pl
from jax.experimental.pallas import tpu as pltpu
from jax.experimental.pallas import tpu_sc as plsc

_H = 128
_NC = 2
_NS = 16
_NW = _NC * _NS
_C = 40
_NH = 5
_ZR = 40

_PERM = []
for _blk in range(_H // 32):
    for _i in range(16):
        _PERM.append(_blk * 32 + _i)
        _PERM.append(_blk * 32 + 16 + _i)
_PERM = tuple(_PERM)


def _matmul_bias_f32_body(x_ref, w_ref, b_ref, o_ref):
    o_ref[...] = (
        jnp.dot(x_ref[...], w_ref[...], preferred_element_type=jnp.float32)
        + b_ref[...]
    )


def _matmul_bias_bf16_body(x_ref, w_ref, b_ref, o_ref):
    o_ref[...] = (
        jnp.dot(x_ref[...], w_ref[...], preferred_element_type=jnp.float32)
        + b_ref[...]
    ).astype(jnp.bfloat16)


def _matmul_bias(x, w, b, out_dtype=jnp.float32, block_n=1000):
    n, din = x.shape
    dout = w.shape[1]
    body = (_matmul_bias_bf16_body if out_dtype == jnp.bfloat16
            else _matmul_bias_f32_body)
    return pl.pallas_call(
        body,
        grid=(n // block_n,),
        in_specs=[
            pl.BlockSpec((block_n, din), lambda i: (i, 0)),
            pl.BlockSpec((din, dout), lambda i: (0, 0)),
            pl.BlockSpec((1, dout), lambda i: (0, 0)),
        ],
        out_specs=pl.BlockSpec((block_n, dout), lambda i: (i, 0)),
        out_shape=jax.ShapeDtypeStruct((n, dout), out_dtype),
    )(x, w, b[None, :])


def _combine2_body(a0, a1, s0, c0, c1, s1, o):
    ya = a0[...] + a1[...] + s0[...]
    yb = c0[...] + c1[...] + s1[...]
    y = jnp.maximum(ya, yb)
    o[...] = jnp.where(y >= 0, y, 0.01 * y)


def _combine1_body(a0, a1, s0, o):
    y = a0[...] + a1[...] + s0[...]
    o[...] = jnp.where(y >= 0, y, 0.01 * y)


def _combine(parts, block_n=1000):
    n = parts[0][0].shape[0]
    args = [a for t in parts for a in t]
    body = _combine2_body if len(parts) == 2 else _combine1_body
    spec = pl.BlockSpec((block_n, _H), lambda i: (i, 0))
    return pl.pallas_call(
        body,
        grid=(n // block_n,),
        in_specs=[spec] * len(args),
        out_specs=spec,
        out_shape=jax.ShapeDtypeStruct((n, _H), jnp.float32),
    )(*args)


def _edge_body(src_ref, dst_ref, k_hbm, q_hbm, v_hbm, out_hbm,
               sidx, didx, kbuf0, kbuf1, kbuf2, kbuf3,
               qbuf0, qbuf1, qbuf2, qbuf3, vbuf0, vbuf1, vbuf2, vbuf3,
               dsc0, dsc1, dsc2, dsc3, acc,
               gsem0, gsem1, gsem2, gsem3,
               isem0, isem1, isem2, isem3,
               ssem0, ssem1, ssem2, ssem3):
    c = lax.axis_index("c")
    s = lax.axis_index("s")
    wid = s * _NC + c
    n_dst = acc.shape[0]
    e_res = sidx.shape[0]
    ch = e_res // _C
    assert ch % 4 == 0 and ch * _C == e_res and ch >= 8
    kbufs = (kbuf0, kbuf1, kbuf2, kbuf3)
    qbufs = (qbuf0, qbuf1, qbuf2, qbuf3)
    vbufs = (vbuf0, vbuf1, vbuf2, vbuf3)
    dscs = (dsc0, dsc1, dsc2, dsc3)
    gsems = (gsem0, gsem1, gsem2, gsem3)
    isems = (isem0, isem1, isem2, isem3)
    ssems = (ssem0, ssem1, ssem2, ssem3)

    def zrow(i, carry):
        for j in range(_H // 16):
            vbuf0[i, pl.ds(j * 16, 16)] = jnp.zeros((16,), jnp.float32)
        return carry
    lax.fori_loop(0, _ZR, zrow, 0)
    n_pieces = n_dst // _ZR
    n_slots = (n_pieces + _NS - 1) // _NS
    for p in range(n_slots):
        pid = p * _NS + s
        r0 = pl.multiple_of(pid * _ZR, 8)

        @pl.when(pid < n_pieces)
        def _():
            pltpu.sync_copy(vbuf0, acc.at[pl.ds(r0, _ZR)])
    plsc.subcore_barrier()

    def stretch(h, hcarry):
        gbase = pl.multiple_of((wid * _NH + h) * e_res, 8)
        pltpu.sync_copy(src_ref.at[pl.ds(gbase, e_res)], sidx)
        pltpu.sync_copy(dst_ref.at[pl.ds(gbase, e_res)], didx)

        def fire(t, sl):
            off = pl.multiple_of(t * _C, 8)
            pltpu.async_copy(
                k_hbm.at[didx.at[pl.ds(off, _C)]], kbufs[sl], gsems[sl])
            pltpu.async_copy(
                q_hbm.at[sidx.at[pl.ds(off, _C)]], qbufs[sl], gsems[sl])
            pltpu.async_copy(
                v_hbm.at[sidx.at[pl.ds(off, _C)]], vbufs[sl], gsems[sl])
            hoff = pl.multiple_of(gbase + t * _C, 8)
            pltpu.async_copy(
                dst_ref.at[pl.ds(hoff, _C)], dscs[sl], isems[sl])

        fire(0, 0)
        fire(1, 1)

        def quad(g, carry):
            t0 = g * 4
            for p in range(4):
                t = t0 + p
                pf = (p + 2) % 4
                tf = t + 2

                @pl.when(tf < ch)
                def _():
                    @pl.when(tf >= 4)
                    def _():
                        pltpu.make_async_copy(
                            v_hbm.at[pl.ds(0, _C)], vbufs[pf],
                            ssems[pf]).wait()
                    fire(tf, pf)

                pltpu.make_async_copy(
                    k_hbm.at[pl.ds(0, _C)], kbufs[p], gsems[p]).wait()
                pltpu.make_async_copy(
                    q_hbm.at[pl.ds(0, _C)], qbufs[p], gsems[p]).wait()
                pltpu.make_async_copy(
                    v_hbm.at[pl.ds(0, _C)], vbufs[p], gsems[p]).wait()
                pltpu.make_async_copy(
                    dst_ref.at[pl.ds(0, _C)], dscs[p], isems[p]).wait()

                kb, qb, vb = kbufs[p], qbufs[p], vbufs[p]

                hi_mask = jnp.full((16,), -65536, jnp.int32)

                @plsc.parallel_loop(0, _C)
                def row(r):
                    for blk in range(_H // 32):
                        c0 = blk * 32
                        kw = kb[r, pl.ds(blk * 16, 16)]
                        qw = qb[r, pl.ds(blk * 16, 16)]
                        k0 = lax.bitcast_convert_type(kw << 16, jnp.float32)
                        k1 = lax.bitcast_convert_type(kw & hi_mask,
                                                      jnp.float32)
                        q0 = lax.bitcast_convert_type(qw << 16, jnp.float32)
                        q1 = lax.bitcast_convert_type(qw & hi_mask,
                                                      jnp.float32)
                        v0 = vb[r, pl.ds(c0, 16)]
                        v1 = vb[r, pl.ds(c0 + 16, 16)]
                        g0 = 1.0 / (1.0 + jnp.exp(-(k0 + q0)))
                        g1 = 1.0 / (1.0 + jnp.exp(-(k1 + q1)))
                        vb[r, pl.ds(c0, 16)] = g0 * v0
                        vb[r, pl.ds(c0 + 16, 16)] = g1 * v1

                pltpu.async_copy(vb, acc.at[dscs[p]], ssems[p], add=True)
            return carry
        lax.fori_loop(0, ch // 4, quad, 0)
        for p in range(4):
            pltpu.make_async_copy(
                v_hbm.at[pl.ds(0, _C)], vbufs[p], ssems[p]).wait()
        return hcarry
    lax.fori_loop(0, _NH, stretch, 0)
    plsc.subcore_barrier()

    for p in range(n_slots):
        pid = p * _NS + s
        r0 = pl.multiple_of(pid * _ZR, 8)

        @pl.when(pid < n_pieces)
        def _():
            pltpu.sync_copy(acc.at[pl.ds(r0, _ZR)], vbuf0)
            pltpu.sync_copy(vbuf0, out_hbm.at[c, pl.ds(r0, _ZR)])


def _edge_pass(src, dst, ktab, qtab, vtab, n_dst):
    ktab = jax.lax.bitcast_convert_type(
        ktab.reshape(ktab.shape[0], _H // 2, 2), jnp.int32)
    qtab = jax.lax.bitcast_convert_type(
        qtab.reshape(qtab.shape[0], _H // 2, 2), jnp.int32)
    e_total = src.shape[0]
    e_res = e_total // (_NW * _NH)
    mesh = plsc.VectorSubcoreMesh(core_axis_name="c", subcore_axis_name="s")
    f = pl.kernel(
        _edge_body,
        out_type=jax.ShapeDtypeStruct((_NC, n_dst, _H), jnp.float32),
        mesh=mesh,
        compiler_params=pltpu.CompilerParams(use_tc_tiling_on_sc=False),
        scratch_types=(
            [pltpu.VMEM((e_res,), jnp.int32)] * 2
            + [pltpu.VMEM((_C, _H // 2), jnp.int32)] * 8
            + [pltpu.VMEM((_C, _H), jnp.float32)] * 4
            + [pltpu.VMEM((_C,), jnp.int32)] * 4
            + [pltpu.VMEM_SHARED((n_dst, _H), jnp.float32)]
            + [pltpu.SemaphoreType.DMA] * 12
        ),
    )
    return f(src, dst, ktab, qtab, vtab)


def kernel(x_user, x_item, edge_buys, edge_bought, edge_follows, params):
    eb = edge_buys.astype(jnp.int32)
    ebb = edge_bought.astype(jnp.int32)
    ef = edge_follows.astype(jnp.int32)
    n_user = x_user.shape[0]
    n_item = x_item.shape[0]
    perm = jnp.asarray(_PERM, jnp.int32)
    xu, xi = x_user, x_item

    for l in range(len(params["layers"])):
        lp = params["layers"][l]
        pb, pf, pbb = lp["buys"], lp["follows"], lp["bought_by"]
        wug = jnp.concatenate(
            [pb["Wq"][:, perm], pf["Wq"][:, perm],
             pf["Wk"][:, perm], pbb["Wk"][:, perm]], axis=1)
        bug = jnp.concatenate(
            [pb["bq"][perm], pf["bq"][perm], pf["bk"][perm], pbb["bk"][perm]])
        wig = jnp.concatenate([pbb["Wq"][:, perm], pb["Wk"][:, perm]], axis=1)
        big = jnp.concatenate([pbb["bq"][perm], pb["bk"][perm]])
        wuv = jnp.concatenate(
            [pb["Wv"], pf["Wv"], pf["Ws"], pbb["Ws"]], axis=1)
        buv = jnp.concatenate([pb["bv"], pf["bv"], pf["b"], pbb["b"]])
        wiv = jnp.concatenate([pbb["Wv"], pb["Ws"]], axis=1)
        biv = jnp.concatenate([pbb["bv"], pb["b"]])

        ug = _matmul_bias(xu, wug, bug, jnp.bfloat16)
        ig = _matmul_bias(xi, wig, big, jnp.bfloat16)
        uv = _matmul_bias(xu, wuv, buv)
        iv = _matmul_bias(xi, wiv, biv)

        q_buys, q_fol = ug[:, 0:128], ug[:, 128:256]
        k_fol, k_bought = ug[:, 256:384], ug[:, 384:512]
        q_bought, k_buys = ig[:, 0:128], ig[:, 128:256]
        v_buys, v_fol = uv[:, 0:128], uv[:, 128:256]
        s_fol, s_bought = uv[:, 256:384], uv[:, 384:512]
        v_bought, s_buys = iv[:, 0:128], iv[:, 128:256]

        agg_buys = _edge_pass(eb[0], eb[1], k_buys, q_buys, v_buys, n_item)
        agg_bought = _edge_pass(
            ebb[0], ebb[1], k_bought, q_bought, v_bought, n_user)
        agg_fol = _edge_pass(ef[0], ef[1], k_fol, q_fol, v_fol, n_user)

        xu = _combine([(agg_bought[0], agg_bought[1], s_bought),
                       (agg_fol[0], agg_fol[1], s_fol)])
        xi = _combine([(agg_buys[0], agg_buys[1], s_buys)])

    wp = jnp.pad(params["lin_W"], ((0, 0), (0, _H - params["lin_W"].shape[1])))
    bp = jnp.pad(params["lin_b"], (0, _H - params["lin_b"].shape[0]))
    out = _matmul_bias(xu, wp, bp)
    return out[:, :params["lin_W"].shape[1]]
